# Initial kernel scaffold; baseline (speedup 1.0000x reference)
import jax
import jax.numpy as jnp
from jax.experimental import pallas as pl


def kernel(inputs, edge_attr, hidden, msg_W1, msg_b1, msg_W2, msg_b2, pres_W1, pres_b1, pres_W2, pres_b2, res_W1, res_b1, res_W2, res_b2, W_ir, b_ir, W_ii, b_ii, W_in, b_in, W_hr, W_hi, W_hh, out_W1, out_b1, out_W2, out_b2, out_W3, out_b3):
    raise NotImplementedError("write your pallas kernel here")


# speedup vs baseline: 2.2024x; 2.2024x over previous
import jax
import jax.numpy as jnp
import numpy as np
from jax.experimental import pallas as pl

N = 64
H = 64
D = 64
FE = 71
E = N * (N - 1)

_send_np, _recv_np = np.where(~np.eye(N, dtype=bool))
_M_pres_np = (_recv_np[None, :] == np.arange(N)[:, None]).astype(np.float32)
_M_msg_np = np.tile(np.eye(N, dtype=np.float32), (1, N))


def _fused_kernel(ea_ref, x_ref, h_ref,
                  mW1_ref, mb1_ref, mW2_ref, mb2_ref,
                  pW1_ref, pb1_ref, pW2_ref, pb2_ref,
                  rW1_ref, rb1_ref, rW2_ref, rb2_ref,
                  Wir_ref, bir_ref, Wii_ref, bii_ref, Win_ref, bin_ref,
                  Whr_ref, Whi_ref, Whh_ref,
                  oW1_ref, ob1_ref, oW2_ref, ob2_ref, oW3_ref, ob3_ref,
                  Mmsg_ref, Mpres_ref,
                  pred_ref, hnew_ref):
    f32 = jnp.float32
    dot = lambda a, b: jnp.dot(a, b, preferred_element_type=f32)

    h = h_ref[0]
    x = x_ref[0]
    ea = ea_ref[0]

    W1r = mW1_ref[0:H, :]
    W1s = mW1_ref[H:2 * H, :]
    A = dot(h, W1r)
    C = dot(h, W1s)
    b1 = mb1_ref[0]
    t1 = jnp.tanh(C[:, None, :] + A[None, :, :] + b1[None, None, :])
    t1 = t1.reshape(N * N, H)
    hm = jnp.tanh(dot(t1, mW2_ref[...]) + mb2_ref[0])
    msg_sum = dot(Mmsg_ref[...], hm)
    t1d = jnp.tanh(A + C + b1[None, :])
    hmd = jnp.tanh(dot(t1d, mW2_ref[...]) + mb2_ref[0])
    hidden_node_emb = (msg_sum - hmd) * (1.0 / (N - 1))

    pm = jax.nn.relu(dot(ea, pW1_ref[...]) + pb1_ref[0])
    pm = jax.nn.relu(dot(pm, pW2_ref[...]) + pb2_ref[0])
    present = dot(Mpres_ref[...], pm) * (1.0 / (N - 1))

    res = jax.nn.relu(dot(x, rW1_ref[...]) + rb1_ref[0])
    res = jax.nn.relu(dot(res, rW2_ref[...]) + rb2_ref[0])
    present = present + res

    r = jax.nn.sigmoid(dot(present, Wir_ref[...]) + bir_ref[0] + dot(hidden_node_emb, Whr_ref[...]))
    i = jax.nn.sigmoid(dot(present, Wii_ref[...]) + bii_ref[0] + dot(hidden_node_emb, Whi_ref[...]))
    n = jnp.tanh(dot(present, Win_ref[...]) + bin_ref[0] + r * dot(hidden_node_emb, Whh_ref[...]))
    h_new = (1.0 - i) * n + i * h

    o = jax.nn.relu(dot(h_new, oW1_ref[...]) + ob1_ref[0])
    o = jax.nn.relu(dot(o, oW2_ref[...]) + ob2_ref[0])
    pred = dot(o, oW3_ref[...]) + ob3_ref[0]

    pred_ref[0] = pred
    hnew_ref[0] = h_new


def kernel(inputs, edge_attr, hidden, msg_W1, msg_b1, msg_W2, msg_b2,
           pres_W1, pres_b1, pres_W2, pres_b2, res_W1, res_b1, res_W2, res_b2,
           W_ir, b_ir, W_ii, b_ii, W_in, b_in, W_hr, W_hi, W_hh,
           out_W1, out_b1, out_W2, out_b2, out_W3, out_b3):
    B = inputs.shape[0]
    M_msg = jnp.asarray(_M_msg_np)
    M_pres = jnp.asarray(_M_pres_np)
    r2 = lambda v: v.reshape(1, -1)

    blk = lambda shape: pl.BlockSpec(shape, lambda b: (b, 0, 0))
    full = lambda a: pl.BlockSpec(a.shape, lambda b: tuple(0 for _ in a.shape))

    args = (edge_attr, inputs, hidden,
            msg_W1, r2(msg_b1), msg_W2, r2(msg_b2),
            pres_W1, r2(pres_b1), pres_W2, r2(pres_b2),
            res_W1, r2(res_b1), res_W2, r2(res_b2),
            W_ir, r2(b_ir), W_ii, r2(b_ii), W_in, r2(b_in),
            W_hr, W_hi, W_hh,
            out_W1, r2(out_b1), out_W2, r2(out_b2), out_W3, r2(out_b3),
            M_msg, M_pres)
    in_specs = [blk((1, E, FE)), blk((1, N, D)), blk((1, N, H))] + \
               [full(a) for a in args[3:]]

    pred, h_new = pl.pallas_call(
        _fused_kernel,
        grid=(B,),
        in_specs=in_specs,
        out_specs=[blk((1, N, D)), blk((1, N, H))],
        out_shape=[jax.ShapeDtypeStruct((B, N, D), jnp.float32),
                   jax.ShapeDtypeStruct((B, N, H), jnp.float32)],
    )(*args)
    return (pred, h_new)
